# baseline (speedup 1.0000x reference)
import functools

import jax
import jax.numpy as jnp
from jax import lax
from jax.experimental import pallas as pl
from jax.experimental.pallas import tpu as pltpu
from jax.experimental.pallas import tpu_sc as plsc

B = 4
N = 10000
D = 128
E = 320000

NC = 2
NT = 16
EPT = E // NT
CH = 80
NCHUNK = EPT // CH
RPT = 640
N_PAD = NT * RPT
NV = D // 16

_mesh = plsc.VectorSubcoreMesh(
    core_axis_name="c", subcore_axis_name="s", num_cores=NC, num_subcores=NT
)


@functools.partial(
    pl.kernel,
    out_type=jax.ShapeDtypeStruct((B, N, D), jnp.float32),
    mesh=_mesh,
    scratch_types=[
        pltpu.VMEM((4, CH), jnp.int32),
        pltpu.VMEM((4, CH), jnp.int32),
        pltpu.VMEM((4, CH), jnp.float32),
        pltpu.VMEM((2, CH, D), jnp.float32),
        pltpu.VMEM_SHARED((N_PAD, D), jnp.float32),
        pltpu.SemaphoreType.DMA,
        pltpu.SemaphoreType.DMA,
    ],
)
def _aggregate(emb_hbm, col_hbm, row_hbm, val_hbm, out_hbm,
               colv, rowv, valv, rows, acc, sem_g, sem_i):
    c = lax.axis_index("c")
    s = lax.axis_index("s")

    zvec = jnp.zeros((16,), jnp.float32)

    def zero_row(e, carry):
        for q in range(NV):
            rows[0, e, pl.ds(q * 16, 16)] = zvec
        return carry

    lax.fori_loop(0, CH, zero_row, 0)

    def idx_fetch(b, j, slot, sync=False):
        copy = pltpu.sync_copy if sync else (
            lambda src, dst: pltpu.async_copy(src, dst, sem_i))
        copy(col_hbm.at[b, s, j], colv.at[slot])
        copy(row_hbm.at[b, s, j], rowv.at[slot])
        copy(val_hbm.at[b, s, j], valv.at[slot])

    def idx_wait(b, j, slot):
        pltpu.make_async_copy(col_hbm.at[b, s, j], colv.at[slot], sem_i).wait()
        pltpu.make_async_copy(row_hbm.at[b, s, j], rowv.at[slot], sem_i).wait()
        pltpu.make_async_copy(val_hbm.at[b, s, j], valv.at[slot], sem_i).wait()

    for i in range(B // NC):
        b = c * (B // NC) + i

        for k in range(RPT // CH):
            pltpu.sync_copy(rows.at[0], acc.at[pl.ds(s * RPT + k * CH, CH)])

        plsc.subcore_barrier()

        idx_fetch(b, 0, 0, sync=True)
        pltpu.async_copy(emb_hbm.at[colv.at[0]], rows.at[0], sem_g)
        idx_fetch(b, 1, 1)

        def chunk(j, carry):
            p = j & 1
            pltpu.make_async_copy(
                emb_hbm.at[colv.at[j & 3]], rows.at[p], sem_g).wait()

            @pl.when(j + 1 < NCHUNK)
            def _start_next():
                nslot = (j + 1) & 3
                idx_wait(b, j + 1, nslot)
                pltpu.async_copy(
                    emb_hbm.at[colv.at[nslot]], rows.at[1 - p], sem_g)

            @pl.when(j + 2 < NCHUNK)
            def _prefetch_idx():
                idx_fetch(b, j + 2, (j + 2) & 3)

            def grp(g, gcarry):
                v16 = valv[j & 3, pl.ds(g * 16, 16)]
                for k in range(16):
                    e = g * 16 + k
                    v = v16[k]
                    for q in range(NV):
                        sl = pl.ds(q * 16, 16)
                        rows[p, e, sl] = rows[p, e, sl] * v
                return gcarry

            lax.fori_loop(0, CH // 16, grp, 0)

            pltpu.sync_copy(rows.at[p], acc.at[rowv.at[j & 3]], add=True)
            return carry

        lax.fori_loop(0, NCHUNK, chunk, 0)

        plsc.subcore_barrier()

        @pl.when(s < NT - 1)
        def _write_full():
            sl = pl.ds(s * RPT, RPT)
            pltpu.sync_copy(acc.at[sl], out_hbm.at[b, sl])

        @pl.when(s == NT - 1)
        def _write_tail():
            sl = pl.ds((NT - 1) * RPT, N - (NT - 1) * RPT)
            pltpu.sync_copy(acc.at[sl], out_hbm.at[b, sl])

        plsc.subcore_barrier()

        if i + 1 < B // NC:
            lax.fori_loop(0, CH, zero_row, 0)


def kernel(last_embs, edge_index, edge_values):
    ei = edge_index.astype(jnp.int32)
    col = ei[:, 1, :] + (jnp.arange(B, dtype=jnp.int32) * N)[:, None]
    row = ei[:, 0, :]
    emb2 = last_embs.reshape(B * N, D)
    col4 = col.reshape(B, NT, NCHUNK, CH)
    row4 = row.reshape(B, NT, NCHUNK, CH)
    val4 = edge_values.reshape(B, NT, NCHUNK, CH)
    return _aggregate(emb2, col4, row4, val4)


# speedup vs baseline: 1.1794x; 1.1794x over previous
import functools

import jax
import jax.numpy as jnp
from jax import lax
from jax.experimental import pallas as pl
from jax.experimental.pallas import tpu as pltpu
from jax.experimental.pallas import tpu_sc as plsc

B = 4
N = 10000
D = 128
E = 320000

NC = 2
NT = 16
EPT = E // NT
CH = 80
NCHUNK = EPT // CH
RPT = 640
N_PAD = NT * RPT
NV = D // 16

_mesh = plsc.VectorSubcoreMesh(
    core_axis_name="c", subcore_axis_name="s", num_cores=NC, num_subcores=NT
)


@functools.partial(
    pl.kernel,
    out_type=jax.ShapeDtypeStruct((B, N, D), jnp.float32),
    mesh=_mesh,
    scratch_types=[
        pltpu.VMEM((NCHUNK, CH), jnp.float32),
        pltpu.VMEM((CH,), jnp.int32),
        pltpu.VMEM((CH,), jnp.int32),
        pltpu.VMEM((CH, D), jnp.float32),
        pltpu.VMEM_SHARED((N_PAD, D), jnp.float32),
        pltpu.SemaphoreType.DMA,
    ],
)
def _aggregate(emb_hbm, col_hbm, row_hbm, val_hbm, out_hbm,
               valv, colv, rowv, rows, acc, sem):
    c = lax.axis_index("c")
    s = lax.axis_index("s")

    zvec = jnp.zeros((16,), jnp.float32)

    def zero_row(e, carry):
        for q in range(NV):
            rows[e, pl.ds(q * 16, 16)] = zvec
        return carry

    lax.fori_loop(0, CH, zero_row, 0)

    for i in range(B // NC):
        b = c * (B // NC) + i

        for k in range(RPT // CH):
            pltpu.sync_copy(rows, acc.at[pl.ds(s * RPT + k * CH, CH)])

        pltpu.sync_copy(val_hbm.at[b, s], valv)

        plsc.subcore_barrier()

        def chunk(j, carry):
            pltpu.sync_copy(col_hbm.at[b, s, j], colv)
            pltpu.sync_copy(row_hbm.at[b, s, j], rowv)
            pltpu.async_copy(emb_hbm.at[colv], rows, sem).wait()

            def grp(g, gcarry):
                v16 = valv[j, pl.ds(g * 16, 16)]
                for k in range(16):
                    e = g * 16 + k
                    vecs = [rows[e, pl.ds(q * 16, 16)] for q in range(NV)]
                    v = v16[k]
                    for q in range(NV):
                        rows[e, pl.ds(q * 16, 16)] = vecs[q] * v
                return gcarry

            lax.fori_loop(0, CH // 16, grp, 0)

            pltpu.sync_copy(rows, acc.at[rowv], add=True)
            return carry

        lax.fori_loop(0, NCHUNK, chunk, 0)

        plsc.subcore_barrier()

        @pl.when(s < NT - 1)
        def _write_full():
            sl = pl.ds(s * RPT, RPT)
            pltpu.sync_copy(acc.at[sl], out_hbm.at[b, sl])

        @pl.when(s == NT - 1)
        def _write_tail():
            sl = pl.ds((NT - 1) * RPT, N - (NT - 1) * RPT)
            pltpu.sync_copy(acc.at[sl], out_hbm.at[b, sl])

        plsc.subcore_barrier()

        if i + 1 < B // NC:
            lax.fori_loop(0, CH, zero_row, 0)


def kernel(last_embs, edge_index, edge_values):
    ei = edge_index.astype(jnp.int32)
    col = ei[:, 1, :] + (jnp.arange(B, dtype=jnp.int32) * N)[:, None]
    row = ei[:, 0, :]
    emb2 = last_embs.reshape(B * N, D)
    col4 = col.reshape(B, NT, NCHUNK, CH)
    row4 = row.reshape(B, NT, NCHUNK, CH)
    val4 = edge_values.reshape(B, NT, NCHUNK, CH)
    return _aggregate(emb2, col4, row4, val4)


# speedup vs baseline: 2.6713x; 2.2649x over previous
import functools

import jax
import jax.numpy as jnp
from jax import lax
from jax.experimental import pallas as pl
from jax.experimental.pallas import tpu as pltpu
from jax.experimental.pallas import tpu_sc as plsc

B = 4
N = 10000
D = 128
E = 320000

NC = 2
NT = 16
EPT = E // NT
CH = 80
NCHUNK = EPT // CH
NPAIR = NCHUNK // 2
RPT = 640
N_PAD = NT * RPT
NV = D // 16

_mesh = plsc.VectorSubcoreMesh(
    core_axis_name="c", subcore_axis_name="s", num_cores=NC, num_subcores=NT
)


@functools.partial(
    pl.kernel,
    out_type=jax.ShapeDtypeStruct((B, N, D), jnp.float32),
    mesh=_mesh,
    scratch_types=[
        pltpu.VMEM((4, CH), jnp.int32),
        pltpu.VMEM((4, CH), jnp.int32),
        pltpu.VMEM((4, CH), jnp.float32),
        pltpu.VMEM((CH, D), jnp.float32),
        pltpu.VMEM((CH, D), jnp.float32),
        pltpu.VMEM_SHARED((N_PAD, D), jnp.float32),
        pltpu.SemaphoreType.DMA,
        pltpu.SemaphoreType.DMA,
        pltpu.SemaphoreType.DMA,
    ],
)
def _aggregate(emb_hbm, col_hbm, row_hbm, val_hbm, out_hbm,
               colv, rowv, valv, rows0, rows1, acc, sem_g, sem_i, sem_s):
    c = lax.axis_index("c")
    s = lax.axis_index("s")
    zvec = jnp.zeros((16,), jnp.float32)

    def zero_rows0(e, carry):
        for q in range(NV):
            rows0[e, pl.ds(q * 16, 16)] = zvec
        return carry

    def idx_fetch(b, j, sync=False):
        slot = j & 3
        copy = pltpu.sync_copy if sync else (
            lambda src, dst: pltpu.async_copy(src, dst, sem_i))
        copy(col_hbm.at[b, s, j], colv.at[slot])
        copy(row_hbm.at[b, s, j], rowv.at[slot])
        copy(val_hbm.at[b, s, j], valv.at[slot])

    def idx_wait(b, j):
        slot = j & 3
        pltpu.make_async_copy(col_hbm.at[b, s, j], colv.at[slot], sem_i).wait()
        pltpu.make_async_copy(row_hbm.at[b, s, j], rowv.at[slot], sem_i).wait()
        pltpu.make_async_copy(val_hbm.at[b, s, j], valv.at[slot], sem_i).wait()

    def gather_start(j, rows_ref):
        pltpu.async_copy(emb_hbm.at[colv.at[j & 3]], rows_ref, sem_g)

    def gather_wait(j, rows_ref):
        pltpu.make_async_copy(emb_hbm.at[colv.at[j & 3]], rows_ref, sem_g).wait()

    def scatter_start(j, rows_ref):
        pltpu.async_copy(rows_ref, acc.at[rowv.at[j & 3]], sem_s, add=True)

    def scatter_wait(j, rows_ref):
        pltpu.make_async_copy(rows_ref, acc.at[rowv.at[j & 3]], sem_s).wait()

    def scale(rows_ref, slot):
        def grp(g, gcarry):
            v16 = valv[slot, pl.ds(g * 16, 16)]
            for k in range(16):
                e = g * 16 + k
                vecs = [rows_ref[e, pl.ds(q * 16, 16)] for q in range(NV)]
                v = v16[k]
                for q in range(NV):
                    rows_ref[e, pl.ds(q * 16, 16)] = vecs[q] * v
            return gcarry

        lax.fori_loop(0, CH // 16, grp, 0)

    def per_graph(i, carry):
        b = c * (B // NC) + i

        lax.fori_loop(0, CH, zero_rows0, 0)
        for k in range(RPT // CH):
            pltpu.sync_copy(rows0, acc.at[pl.ds(s * RPT + k * CH, CH)])

        plsc.subcore_barrier()

        idx_fetch(b, 0, sync=True)
        gather_start(0, rows0)
        idx_fetch(b, 1)

        def pair(t, pcarry):
            j0 = 2 * t
            j1 = j0 + 1

            gather_wait(j0, rows0)
            idx_wait(b, j1)

            @pl.when(t > 0)
            def _wait_prev_odd_scatter():
                scatter_wait(j1 - 2, rows1)

            gather_start(j1, rows1)

            @pl.when(j0 + 2 < NCHUNK)
            def _prefetch_even():
                idx_fetch(b, j0 + 2)

            scale(rows0, j0 & 3)
            scatter_start(j0, rows0)

            gather_wait(j1, rows1)

            @pl.when(j0 + 2 < NCHUNK)
            def _next_even_gather():
                idx_wait(b, j0 + 2)
                scatter_wait(j0, rows0)
                gather_start(j0 + 2, rows0)

            @pl.when(j1 + 2 < NCHUNK)
            def _prefetch_odd():
                idx_fetch(b, j1 + 2)

            scale(rows1, j1 & 3)
            scatter_start(j1, rows1)
            return pcarry

        lax.fori_loop(0, NPAIR, pair, 0)

        scatter_wait(NCHUNK - 2, rows0)
        scatter_wait(NCHUNK - 1, rows1)

        plsc.subcore_barrier()

        @pl.when(s < NT - 1)
        def _write_full():
            sl = pl.ds(s * RPT, RPT)
            pltpu.sync_copy(acc.at[sl], out_hbm.at[b, sl])

        @pl.when(s == NT - 1)
        def _write_tail():
            sl = pl.ds((NT - 1) * RPT, N - (NT - 1) * RPT)
            pltpu.sync_copy(acc.at[sl], out_hbm.at[b, sl])

        plsc.subcore_barrier()
        return carry

    lax.fori_loop(0, B // NC, per_graph, 0)


def kernel(last_embs, edge_index, edge_values):
    ei = edge_index.astype(jnp.int32)
    col = ei[:, 1, :] + (jnp.arange(B, dtype=jnp.int32) * N)[:, None]
    row = ei[:, 0, :]
    emb2 = last_embs.reshape(B * N, D)
    col4 = col.reshape(B, NT, NCHUNK, CH)
    row4 = row.reshape(B, NT, NCHUNK, CH)
    val4 = edge_values.reshape(B, NT, NCHUNK, CH)
    return _aggregate(emb2, col4, row4, val4)


# speedup vs baseline: 3.4981x; 1.3095x over previous
import functools

import jax
import jax.numpy as jnp
from jax import lax
from jax.experimental import pallas as pl
from jax.experimental.pallas import tpu as pltpu
from jax.experimental.pallas import tpu_sc as plsc

B = 4
N = 10000
D = 128
E = 320000

NC = 2
NT = 16
EPT = E // NT
CH = 80
NCHUNK = EPT // CH
RPT = 640
N_PAD = NT * RPT
NV = D // 16

_mesh = plsc.VectorSubcoreMesh(
    core_axis_name="c", subcore_axis_name="s", num_cores=NC, num_subcores=NT
)


@functools.partial(
    pl.kernel,
    out_type=jax.ShapeDtypeStruct((B, N, D), jnp.float32),
    mesh=_mesh,
    scratch_types=[
        pltpu.VMEM((8, CH), jnp.int32),
        pltpu.VMEM((8, CH), jnp.int32),
        pltpu.VMEM((8, CH), jnp.float32),
        pltpu.VMEM((4 * CH, D), jnp.float32),
        pltpu.VMEM_SHARED((N_PAD, D), jnp.float32),
        pltpu.SemaphoreType.DMA,
        pltpu.SemaphoreType.DMA,
        pltpu.SemaphoreType.DMA,
    ],
)
def _aggregate(emb_hbm, col_hbm, row_hbm, val_hbm, out_hbm,
               colv, rowv, valv, rows, acc, sem_g, sem_i, sem_s):
    c = lax.axis_index("c")
    s = lax.axis_index("s")
    zvec = jnp.zeros((16,), jnp.float32)

    def zero_row(e, carry):
        for q in range(NV):
            rows[e, pl.ds(q * 16, 16)] = zvec
        return carry

    def idx_fetch(b, j, sync=False):
        slot = j & 7
        copy = pltpu.sync_copy if sync else (
            lambda src, dst: pltpu.async_copy(src, dst, sem_i))
        copy(col_hbm.at[b, s, j], colv.at[slot])
        copy(row_hbm.at[b, s, j], rowv.at[slot])
        copy(val_hbm.at[b, s, j], valv.at[slot])

    def idx_wait(b, j):
        slot = j & 7
        pltpu.make_async_copy(col_hbm.at[b, s, j], colv.at[slot], sem_i).wait()
        pltpu.make_async_copy(row_hbm.at[b, s, j], rowv.at[slot], sem_i).wait()
        pltpu.make_async_copy(val_hbm.at[b, s, j], valv.at[slot], sem_i).wait()

    def buf(j):
        return rows.at[pl.ds((j & 3) * CH, CH)]

    def gather_start(j):
        pltpu.async_copy(emb_hbm.at[colv.at[j & 7]], buf(j), sem_g)

    def gather_wait(j):
        pltpu.make_async_copy(emb_hbm.at[colv.at[j & 7]], buf(j), sem_g).wait()

    def scatter_start(j):
        pltpu.async_copy(buf(j), acc.at[rowv.at[j & 7]], sem_s, add=True)

    def scatter_wait(j):
        pltpu.make_async_copy(buf(j), acc.at[rowv.at[j & 7]], sem_s).wait()

    def scale(j):
        slot = j & 7
        base = (j & 3) * CH

        def grp(g, gcarry):
            v16 = valv[slot, pl.ds(g * 16, 16)]
            e0 = base + g * 16
            for k in range(16):
                e = e0 + k
                vecs = [rows[e, pl.ds(q * 16, 16)] for q in range(NV)]
                v = v16[k]
                for q in range(NV):
                    rows[e, pl.ds(q * 16, 16)] = vecs[q] * v
            return gcarry

        lax.fori_loop(0, CH // 16, grp, 0)

    def per_graph(i, carry):
        b = c * (B // NC) + i

        lax.fori_loop(0, CH, zero_row, 0)
        for k in range(RPT // CH):
            pltpu.sync_copy(rows.at[pl.ds(0, CH)],
                            acc.at[pl.ds(s * RPT + k * CH, CH)])

        plsc.subcore_barrier()

        idx_fetch(b, 0, sync=True)
        idx_fetch(b, 1, sync=True)
        gather_start(0)
        gather_start(1)
        idx_fetch(b, 2)

        def chunk(j, ccarry):
            gather_wait(j)

            @pl.when(j >= 2)
            def _free_next_buf():
                scatter_wait(j - 2)

            @pl.when(j + 2 < NCHUNK)
            def _next_gather():
                idx_wait(b, j + 2)
                gather_start(j + 2)

            @pl.when(j + 3 < NCHUNK)
            def _prefetch_idx():
                idx_fetch(b, j + 3)

            scale(j)
            scatter_start(j)
            return ccarry

        lax.fori_loop(0, NCHUNK, chunk, 0)

        scatter_wait(NCHUNK - 2)
        scatter_wait(NCHUNK - 1)

        plsc.subcore_barrier()

        @pl.when(s < NT - 1)
        def _write_full():
            sl = pl.ds(s * RPT, RPT)
            pltpu.sync_copy(acc.at[sl], out_hbm.at[b, sl])

        @pl.when(s == NT - 1)
        def _write_tail():
            sl = pl.ds((NT - 1) * RPT, N - (NT - 1) * RPT)
            pltpu.sync_copy(acc.at[sl], out_hbm.at[b, sl])

        plsc.subcore_barrier()
        return carry

    lax.fori_loop(0, B // NC, per_graph, 0)


def kernel(last_embs, edge_index, edge_values):
    ei = edge_index.astype(jnp.int32)
    col = ei[:, 1, :] + (jnp.arange(B, dtype=jnp.int32) * N)[:, None]
    row = ei[:, 0, :]
    emb2 = last_embs.reshape(B * N, D)
    col4 = col.reshape(B, NT, NCHUNK, CH)
    row4 = row.reshape(B, NT, NCHUNK, CH)
    val4 = edge_values.reshape(B, NT, NCHUNK, CH)
    return _aggregate(emb2, col4, row4, val4)
